# baseline (speedup 1.0000x reference)
import jax
import jax.numpy as jnp
from jax import lax
from jax.experimental import pallas as pl
from jax.experimental.pallas import tpu as pltpu
from jax.experimental.pallas import tpu_sc as plsc

N_NODES = 100000
BATCH = 16384
LAMB_LEN = 4
MSG_DIM = 64
ROW = LAMB_LEN * MSG_DIM

NC, NS, L = 2, 16, 16
NW = NC * NS
RNG = 3128
LAST_RNG = N_NODES - (NW - 1) * RNG
N_PAD = NW * RNG
CAP = 4096
NG_IN = BATCH // L
BIG = 1 << 30

CH = 128
NCH = (RNG + CH - 1) // CH


def _body(ids_hbm, msgs_hbm, ts_hbm, mem_hbm, lu_hbm, lam_hbm,
          out_mem, out_lu,
          ids_v, ts_v, cb_v, tab_v, lu_v, lam_v, shift_v,
          idxw_v, bidx_v, idxw2_v, bidx2_v, idxc0_v, idxc1_v,
          rbuf, mbuf, cp0, cp1,
          sgi0, sgi1, sgo0, sgo1, sso0, sso1):
    wid = lax.axis_index("s") * NC + lax.axis_index("c")
    lo = wid * RNG
    iota = lax.iota(jnp.int32, L)

    pltpu.sync_copy(ids_hbm, ids_v)
    pltpu.sync_copy(ts_hbm, ts_v)
    pltpu.sync_copy(lam_hbm, lam_v)
    pltpu.sync_copy(lu_hbm.at[pl.ds(lo, RNG)], lu_v)
    shift_v[pl.ds(L, L)] = jnp.full((L,), BIG, jnp.int32)

    own_max = jnp.where(wid == NW - 1, LAST_RNG, RNG) - 1
    bufs = (cp0, cp1)
    idxs = (idxc0_v, idxc1_v)
    sin = (sgi0, sgi1)
    sout = (sgo0, sgo1)

    def build_idx(c, s):
        for q in range(CH // L):
            idxs[s][pl.ds(q * L, L)] = lo + jnp.minimum(
                c * CH + q * L + iota, own_max)

    build_idx(0, 0)
    din = [pltpu.async_copy(mem_hbm.at[idxc0_v], cp0, sgi0), None]
    dout = [None, None]
    for c in range(NCH):
        s = c % 2
        ns = (c + 1) % 2
        if c + 1 < NCH:
            if dout[ns] is not None:
                dout[ns].wait()
            build_idx(c + 1, ns)
            din[ns] = pltpu.async_copy(mem_hbm.at[idxs[ns]], bufs[ns],
                                       sin[ns])
        din[s].wait()
        dout[s] = pltpu.async_copy(bufs[s], out_mem.at[idxs[s]], sout[s])
    dout[(NCH - 1) % 2].wait()
    dout[NCH % 2].wait()

    def scan_one(g, off):
        idv = ids_v[pl.ds(g * L, L)]
        m = (idv >= lo) & (idv < lo + RNG)
        ps = plsc.cumsum(m.astype(jnp.int32))
        pos = jnp.maximum(off + ps - 1, 0)
        m = m & (pos < CAP)
        plsc.store_scatter(cb_v, [pos], iota + g * L, mask=m)
        return off + plsc.all_reduce_population_count(m)[0]

    def scan_body(h, off):
        off = scan_one(h * 2, off)
        return scan_one(h * 2 + 1, off)

    n = lax.fori_loop(0, NG_IN // 2, scan_body, jnp.int32(0))
    n_groups = (n + L - 1) // L

    def win_scatter(g, carry):
        valid = (iota + g * L) < n
        bv = jnp.where(valid, cb_v[pl.ds(g * L, L)], 0)
        idv = plsc.load_gather(ids_v, [bv])
        r = idv - lo
        key = jnp.where(valid, r * L + iota, BIG)
        bv = jnp.where(valid, bv, -1)
        k_s, b_s = plsc.sort_key_val(key, bv)
        shift_v[pl.ds(0, L)] = k_s
        k_next = plsc.load_gather(shift_v, [iota + 1])
        r_s = jnp.right_shift(k_s, 4)
        is_last = (r_s != jnp.right_shift(k_next, 4)) & (k_s < BIG)
        plsc.store_scatter(tab_v, [r_s], b_s, mask=is_last)
        return carry

    lax.fori_loop(0, n_groups, win_scatter, jnp.int32(0))

    def win_compact(g, off):
        valid = (iota + g * L) < n
        bv = jnp.where(valid, cb_v[pl.ds(g * L, L)], 0)
        idv = plsc.load_gather(ids_v, [bv])
        r = jnp.where(valid, idv - lo, 0)
        w = plsc.load_gather(tab_v, [r])
        win = valid & (w == bv)
        ps = plsc.cumsum(win.astype(jnp.int32))
        pos = jnp.maximum(off + ps - 1, 0)
        plsc.store_scatter(cb_v, [pos], bv, mask=win)
        return off + plsc.all_reduce_population_count(win)[0]

    n_u = lax.fori_loop(0, n_groups, win_compact, jnp.int32(0))
    n_ugroups = (n_u + L - 1) // L

    def fetch(g, ixw, ixb, rb, mb, s1, s2):
        pos = jnp.minimum(iota + g * L, n_u - 1)
        bv = plsc.load_gather(cb_v, [pos])
        idv = plsc.load_gather(ids_v, [bv])
        tsv = plsc.load_gather(ts_v, [bv])
        luv = plsc.load_gather(lu_v, [idv - lo])
        ixw[pl.ds(0, L)] = idv
        ixb[pl.ds(0, L)] = bv
        dm = pltpu.async_copy(mem_hbm.at[ixw], rb, s1)
        dg = pltpu.async_copy(msgs_hbm.at[ixb], mb, s2)
        return luv - tsv, dm, dg

    def compute(td, rb, mb):
        for l in range(LAMB_LEN):
            dl = jnp.exp(td / lam_v[pl.ds(l * L, L)])

            def pos_body(i, c, dl=dl, l=l):
                base = l * MSG_DIM + i * 8
                for k in range(8):
                    fp = jnp.zeros((L,), jnp.int32) + (base + k)
                    mv = plsc.load_gather(rb, [iota, fp])
                    gv = plsc.load_gather(mb, [iota, fp])
                    plsc.store_scatter(rb, [iota, fp], gv + dl * mv)
                return c

            lax.fori_loop(0, MSG_DIM // 8, pos_body, jnp.int32(0))

    rbuf2 = cp0.at[pl.ds(0, L)]
    mbuf2 = cp0.at[pl.ds(L, L)]

    def upd2(g2, carry):
        tdA, dA1, dA2 = fetch(g2 * 2, idxw_v, bidx_v, rbuf, mbuf, sgi0, sgi1)
        tdB, dB1, dB2 = fetch(g2 * 2 + 1, idxw2_v, bidx2_v, rbuf2, mbuf2,
                              sgo0, sgo1)
        dA1.wait()
        dA2.wait()
        compute(tdA, rbuf, mbuf)
        oA = pltpu.async_copy(rbuf, out_mem.at[idxw_v], sso0)
        dB1.wait()
        dB2.wait()
        compute(tdB, rbuf2, mbuf2)
        oB = pltpu.async_copy(rbuf2, out_mem.at[idxw2_v], sso1)
        oA.wait()
        oB.wait()
        return carry

    lax.fori_loop(0, (n_ugroups + 1) // 2, upd2, jnp.int32(0))

    def lu_scatter(g, carry):
        pos = jnp.minimum(iota + g * L, n_u - 1)
        bv = plsc.load_gather(cb_v, [pos])
        idv = plsc.load_gather(ids_v, [bv])
        tsv = plsc.load_gather(ts_v, [bv])
        plsc.store_scatter(lu_v, [idv - lo], tsv)
        return carry

    lax.fori_loop(0, n_ugroups, lu_scatter, jnp.int32(0))
    pltpu.sync_copy(lu_v, out_lu.at[pl.ds(lo, RNG)])


@jax.jit
def _run(ids, msgs2, ts, mem2, lu_p, lam_p):
    f = pl.kernel(
        _body,
        out_type=(
            jax.ShapeDtypeStruct((N_NODES, ROW), jnp.float32),
            jax.ShapeDtypeStruct((N_PAD,), jnp.float32),
        ),
        mesh=plsc.VectorSubcoreMesh(core_axis_name="c", subcore_axis_name="s"),
        compiler_params=pltpu.CompilerParams(needs_layout_passes=False),
        scratch_types=[
            pltpu.VMEM((BATCH,), jnp.int32),
            pltpu.VMEM((BATCH,), jnp.float32),
            pltpu.VMEM((CAP,), jnp.int32),
            pltpu.VMEM((RNG + 8,), jnp.int32),
            pltpu.VMEM((RNG,), jnp.float32),
            pltpu.VMEM((LAMB_LEN * L,), jnp.float32),
            pltpu.VMEM((2 * L,), jnp.int32),
            pltpu.VMEM((L,), jnp.int32),
            pltpu.VMEM((L,), jnp.int32),
            pltpu.VMEM((L,), jnp.int32),
            pltpu.VMEM((L,), jnp.int32),
            pltpu.VMEM((CH,), jnp.int32),
            pltpu.VMEM((CH,), jnp.int32),
            pltpu.VMEM((L, ROW), jnp.float32),
            pltpu.VMEM((L, ROW), jnp.float32),
            pltpu.VMEM((CH, ROW), jnp.float32),
            pltpu.VMEM((CH, ROW), jnp.float32),
            pltpu.SemaphoreType.DMA,
            pltpu.SemaphoreType.DMA,
            pltpu.SemaphoreType.DMA,
            pltpu.SemaphoreType.DMA,
            pltpu.SemaphoreType.DMA,
            pltpu.SemaphoreType.DMA,
        ],
    )
    return f(ids, msgs2, ts, mem2, lu_p, lam_p)


def kernel(unique_node_ids, unique_messages, timestamps, memory, last_update,
           lambs):
    mem2 = memory.reshape(N_NODES, ROW)
    msgs2 = unique_messages.reshape(BATCH, ROW)
    lu_p = jnp.pad(last_update, (0, N_PAD - N_NODES))
    lam_p = jnp.repeat(lambs, L)
    out_mem, out_lu = _run(unique_node_ids, msgs2, timestamps, mem2, lu_p,
                           lam_p)
    return (out_mem.reshape(N_NODES, LAMB_LEN, MSG_DIM), out_lu[:N_NODES])


# speedup vs baseline: 1.0126x; 1.0126x over previous
import jax
import jax.numpy as jnp
from jax import lax
from jax.experimental import pallas as pl
from jax.experimental.pallas import tpu as pltpu
from jax.experimental.pallas import tpu_sc as plsc

N_NODES = 100000
BATCH = 16384
LAMB_LEN = 4
MSG_DIM = 64
ROW = LAMB_LEN * MSG_DIM

NC, NS, L = 2, 16, 16
NW = NC * NS
RNG = 3128
LAST_RNG = N_NODES - (NW - 1) * RNG
N_PAD = NW * RNG
CAP = 4096
NG_IN = BATCH // L
BIG = 1 << 30

CH = 64
NCH = (RNG + CH - 1) // CH


def _body(ids_hbm, msgs_hbm, ts_hbm, mem_hbm, lu_hbm, lam_hbm,
          out_mem, out_lu,
          ids_v, ts_v, cb_v, tab_v, lu_v, lam_v, shift_v,
          idxw_v, bidx_v, idxw2_v, bidx2_v, idxc0_v, idxc1_v,
          rbuf, mbuf, rbuf2, mbuf2, cp0, cp1,
          sgi0, sgi1, sgo0, sgo1, sso0, sso1):
    wid = lax.axis_index("s") * NC + lax.axis_index("c")
    lo = wid * RNG
    iota = lax.iota(jnp.int32, L)

    pltpu.sync_copy(ids_hbm, ids_v)
    pltpu.sync_copy(ts_hbm, ts_v)
    pltpu.sync_copy(lam_hbm, lam_v)
    pltpu.sync_copy(lu_hbm.at[pl.ds(lo, RNG)], lu_v)
    shift_v[pl.ds(L, L)] = jnp.full((L,), BIG, jnp.int32)

    own_max = jnp.where(wid == NW - 1, LAST_RNG, RNG) - 1
    bufs = (cp0, cp1)
    idxs = (idxc0_v, idxc1_v)
    sin = (sgi0, sgi1)
    sout = (sgo0, sgo1)

    def build_idx(c, s):
        for q in range(CH // L):
            idxs[s][pl.ds(q * L, L)] = lo + jnp.minimum(
                c * CH + q * L + iota, own_max)

    build_idx(0, 0)
    din = [pltpu.async_copy(mem_hbm.at[idxc0_v], cp0, sgi0), None]
    dout = [None, None]
    for c in range(NCH):
        s = c % 2
        ns = (c + 1) % 2
        if c + 1 < NCH:
            if dout[ns] is not None:
                dout[ns].wait()
            build_idx(c + 1, ns)
            din[ns] = pltpu.async_copy(mem_hbm.at[idxs[ns]], bufs[ns],
                                       sin[ns])
        din[s].wait()
        dout[s] = pltpu.async_copy(bufs[s], out_mem.at[idxs[s]], sout[s])
    dout[(NCH - 1) % 2].wait()
    dout[NCH % 2].wait()

    def scan_one(g, off):
        idv = ids_v[pl.ds(g * L, L)]
        m = (idv >= lo) & (idv < lo + RNG)
        ps = plsc.cumsum(m.astype(jnp.int32))
        pos = jnp.maximum(off + ps - 1, 0)
        m = m & (pos < CAP)
        plsc.store_scatter(cb_v, [pos], iota + g * L, mask=m)
        return off + plsc.all_reduce_population_count(m)[0]

    def scan_body(h, off):
        off = scan_one(h * 2, off)
        return scan_one(h * 2 + 1, off)

    n = lax.fori_loop(0, NG_IN // 2, scan_body, jnp.int32(0))
    n_groups = (n + L - 1) // L

    def win_scatter(g, carry):
        valid = (iota + g * L) < n
        bv = jnp.where(valid, cb_v[pl.ds(g * L, L)], 0)
        idv = plsc.load_gather(ids_v, [bv])
        r = idv - lo
        key = jnp.where(valid, r * L + iota, BIG)
        bv = jnp.where(valid, bv, -1)
        k_s, b_s = plsc.sort_key_val(key, bv)
        shift_v[pl.ds(0, L)] = k_s
        k_next = plsc.load_gather(shift_v, [iota + 1])
        r_s = jnp.right_shift(k_s, 4)
        is_last = (r_s != jnp.right_shift(k_next, 4)) & (k_s < BIG)
        plsc.store_scatter(tab_v, [r_s], b_s, mask=is_last)
        return carry

    lax.fori_loop(0, n_groups, win_scatter, jnp.int32(0))

    def win_compact(g, off):
        valid = (iota + g * L) < n
        bv = jnp.where(valid, cb_v[pl.ds(g * L, L)], 0)
        idv = plsc.load_gather(ids_v, [bv])
        r = jnp.where(valid, idv - lo, 0)
        w = plsc.load_gather(tab_v, [r])
        win = valid & (w == bv)
        ps = plsc.cumsum(win.astype(jnp.int32))
        pos = jnp.maximum(off + ps - 1, 0)
        plsc.store_scatter(cb_v, [pos], bv, mask=win)
        return off + plsc.all_reduce_population_count(win)[0]

    n_u = lax.fori_loop(0, n_groups, win_compact, jnp.int32(0))
    n_ugroups = (n_u + L - 1) // L

    def fetch(g, ixw, ixb, rb, mb, s1, s2):
        pos = jnp.minimum(iota + g * L, n_u - 1)
        bv = plsc.load_gather(cb_v, [pos])
        idv = plsc.load_gather(ids_v, [bv])
        tsv = plsc.load_gather(ts_v, [bv])
        luv = plsc.load_gather(lu_v, [idv - lo])
        ixw[pl.ds(0, L)] = idv
        ixb[pl.ds(0, L)] = bv
        dm = pltpu.async_copy(mem_hbm.at[ixw], rb, s1)
        dg = pltpu.async_copy(msgs_hbm.at[ixb], mb, s2)
        return luv - tsv, dm, dg

    def compute(td, rb, mb):
        for l in range(LAMB_LEN):
            dl = jnp.exp(td / lam_v[pl.ds(l * L, L)])

            def pos_body(i, c, dl=dl, l=l):
                base = l * MSG_DIM + i * 8
                for k in range(8):
                    fp = jnp.zeros((L,), jnp.int32) + (base + k)
                    mv = plsc.load_gather(rb, [iota, fp])
                    gv = plsc.load_gather(mb, [iota, fp])
                    plsc.store_scatter(rb, [iota, fp], gv + dl * mv)
                return c

            lax.fori_loop(0, MSG_DIM // 8, pos_body, jnp.int32(0))

    def upd2(g2, carry):
        tdA, dA1, dA2 = fetch(g2 * 2, idxw_v, bidx_v, rbuf, mbuf, sgi0, sgi1)
        tdB, dB1, dB2 = fetch(g2 * 2 + 1, idxw2_v, bidx2_v, rbuf2, mbuf2,
                              sgo0, sgo1)
        dA1.wait()
        dA2.wait()
        compute(tdA, rbuf, mbuf)
        oA = pltpu.async_copy(rbuf, out_mem.at[idxw_v], sso0)
        dB1.wait()
        dB2.wait()
        compute(tdB, rbuf2, mbuf2)
        oB = pltpu.async_copy(rbuf2, out_mem.at[idxw2_v], sso1)
        oA.wait()
        oB.wait()
        return carry

    lax.fori_loop(0, (n_ugroups + 1) // 2, upd2, jnp.int32(0))

    def lu_scatter(g, carry):
        pos = jnp.minimum(iota + g * L, n_u - 1)
        bv = plsc.load_gather(cb_v, [pos])
        idv = plsc.load_gather(ids_v, [bv])
        tsv = plsc.load_gather(ts_v, [bv])
        plsc.store_scatter(lu_v, [idv - lo], tsv)
        return carry

    lax.fori_loop(0, n_ugroups, lu_scatter, jnp.int32(0))
    pltpu.sync_copy(lu_v, out_lu.at[pl.ds(lo, RNG)])


@jax.jit
def _run(ids, msgs2, ts, mem2, lu_p, lam_p):
    f = pl.kernel(
        _body,
        out_type=(
            jax.ShapeDtypeStruct((N_NODES, ROW), jnp.float32),
            jax.ShapeDtypeStruct((N_PAD,), jnp.float32),
        ),
        mesh=plsc.VectorSubcoreMesh(core_axis_name="c", subcore_axis_name="s"),
        compiler_params=pltpu.CompilerParams(needs_layout_passes=False),
        scratch_types=[
            pltpu.VMEM((BATCH,), jnp.int32),
            pltpu.VMEM((BATCH,), jnp.float32),
            pltpu.VMEM((CAP,), jnp.int32),
            pltpu.VMEM((RNG + 8,), jnp.int32),
            pltpu.VMEM((RNG,), jnp.float32),
            pltpu.VMEM((LAMB_LEN * L,), jnp.float32),
            pltpu.VMEM((2 * L,), jnp.int32),
            pltpu.VMEM((L,), jnp.int32),
            pltpu.VMEM((L,), jnp.int32),
            pltpu.VMEM((L,), jnp.int32),
            pltpu.VMEM((L,), jnp.int32),
            pltpu.VMEM((CH,), jnp.int32),
            pltpu.VMEM((CH,), jnp.int32),
            pltpu.VMEM((L, ROW), jnp.float32),
            pltpu.VMEM((L, ROW), jnp.float32),
            pltpu.VMEM((L, ROW), jnp.float32),
            pltpu.VMEM((L, ROW), jnp.float32),
            pltpu.VMEM((CH, ROW), jnp.float32),
            pltpu.VMEM((CH, ROW), jnp.float32),
            pltpu.SemaphoreType.DMA,
            pltpu.SemaphoreType.DMA,
            pltpu.SemaphoreType.DMA,
            pltpu.SemaphoreType.DMA,
            pltpu.SemaphoreType.DMA,
            pltpu.SemaphoreType.DMA,
        ],
    )
    return f(ids, msgs2, ts, mem2, lu_p, lam_p)


def kernel(unique_node_ids, unique_messages, timestamps, memory, last_update,
           lambs):
    mem2 = memory.reshape(N_NODES, ROW)
    msgs2 = unique_messages.reshape(BATCH, ROW)
    lu_p = jnp.pad(last_update, (0, N_PAD - N_NODES))
    lam_p = jnp.repeat(lambs, L)
    out_mem, out_lu = _run(unique_node_ids, msgs2, timestamps, mem2, lu_p,
                           lam_p)
    return (out_mem.reshape(N_NODES, LAMB_LEN, MSG_DIM), out_lu[:N_NODES])
